# baseline (speedup 1.0000x reference)
import functools

import jax
import jax.numpy as jnp
from jax import lax
from jax.experimental import pallas as pl
from jax.experimental.pallas import tpu as pltpu
from jax.experimental.pallas import tpu_sc as plsc

_BATCH = 16384
_DIM = 32
_NC = 2
_NS = 16
_NW = _NC * _NS
_ROWS_PER_W = _BATCH // _NW
_GPR = 128 // _DIM
_CHUNK = 128
_NCHUNK = _ROWS_PER_W // _CHUNK

_mesh = plsc.VectorSubcoreMesh(core_axis_name="c", subcore_axis_name="s")


@functools.partial(
    pl.kernel,
    mesh=_mesh,
    out_type=jax.ShapeDtypeStruct((_BATCH, _DIM), jnp.float32),
    scratch_types=[
        pltpu.VMEM((_ROWS_PER_W,), jnp.int32),
        pltpu.VMEM((_NCHUNK, _CHUNK), jnp.int32),
        pltpu.VMEM((_ROWS_PER_W, 128), jnp.float32),
        pltpu.VMEM((_ROWS_PER_W, _DIM), jnp.float32),
        pltpu.SemaphoreType.DMA,
    ],
    compiler_params=pltpu.CompilerParams(
        use_tc_tiling_on_sc=False, needs_layout_passes=False
    ),
)
def _gather_kernel(idx_hbm, table_hbm, out_hbm, idx_v, grp_v, gbuf, rows_v,
                   sem):
    wid = lax.axis_index("s") * _NC + lax.axis_index("c")
    base = wid * _ROWS_PER_W
    pltpu.sync_copy(idx_hbm.at[pl.ds(base, _ROWS_PER_W)], idx_v)

    for j in range(_NCHUNK):
        for k in range(0, _CHUNK, 16):
            v = idx_v[pl.ds(j * _CHUNK + k, 16)]
            grp_v[j, pl.ds(k, 16)] = lax.shift_right_logical(v, 2)

    copies = [
        pltpu.async_copy(
            table_hbm.at[grp_v.at[j]],
            gbuf.at[pl.ds(j * _CHUNK, _CHUNK)],
            sem,
        )
        for j in range(_NCHUNK)
    ]
    for c in copies:
        c.wait()

    for k in range(0, _ROWS_PER_W, 16):
        slot = lax.iota(jnp.int32, 16) + k
        v = idx_v[pl.ds(k, 16)]
        sub = lax.bitwise_and(v, _GPR - 1)
        for c in range(_DIM):
            col = sub * _DIM + c
            vals = plsc.load_gather(gbuf, [slot, col])
            plsc.store_scatter(rows_v, [slot, jnp.full((16,), c, jnp.int32)],
                               vals)

    pltpu.sync_copy(rows_v, out_hbm.at[pl.ds(base, _ROWS_PER_W)])


def kernel(y_true, proxies):
    table2 = proxies.reshape(250000, 128)
    return _gather_kernel(y_true.astype(jnp.int32), table2)


# speedup vs baseline: 1.7387x; 1.7387x over previous
import functools

import jax
import jax.numpy as jnp
from jax import lax
from jax.experimental import pallas as pl
from jax.experimental.pallas import tpu as pltpu
from jax.experimental.pallas import tpu_sc as plsc

_BATCH = 16384
_DIM = 32
_NC = 2
_NS = 16
_NW = _NC * _NS
_ROWS_PER_W = _BATCH // _NW
_NSEM = 4

_mesh = plsc.VectorSubcoreMesh(core_axis_name="c", subcore_axis_name="s")


@functools.partial(
    pl.kernel,
    mesh=_mesh,
    out_type=jax.ShapeDtypeStruct((_BATCH, _DIM), jnp.float32),
    scratch_types=[
        pltpu.VMEM((_ROWS_PER_W,), jnp.int32),
        pltpu.VMEM((_ROWS_PER_W, _DIM), jnp.float32),
        pltpu.SemaphoreType.DMA,
        pltpu.SemaphoreType.DMA,
        pltpu.SemaphoreType.DMA,
        pltpu.SemaphoreType.DMA,
    ],
)
def _gather_kernel(idx_hbm, table_hbm, out_hbm, idx_s, rows_v,
                   sem0, sem1, sem2, sem3):
    sems = (sem0, sem1, sem2, sem3)
    wid = lax.axis_index("s") * _NC + lax.axis_index("c")
    base = wid * _ROWS_PER_W
    pltpu.sync_copy(idx_hbm.at[pl.ds(base, _ROWS_PER_W)], idx_s)

    def fire(c, _):
        vchunk = idx_s[pl.ds(c * 16, 16)]
        for k in range(16):
            pltpu.async_copy(
                table_hbm.at[pl.ds(vchunk[k], 1)],
                rows_v.at[pl.ds(c * 16 + k, 1)],
                sems[k % _NSEM],
            )
        return ()

    lax.fori_loop(0, _ROWS_PER_W // 16, fire, ())
    for q in range(_NSEM):
        pltpu.make_async_copy(
            table_hbm.at[pl.ds(0, _ROWS_PER_W // _NSEM)],
            rows_v.at[pl.ds(0, _ROWS_PER_W // _NSEM)],
            sems[q],
        ).wait()
    pltpu.sync_copy(rows_v, out_hbm.at[pl.ds(base, _ROWS_PER_W)])


def kernel(y_true, proxies):
    return _gather_kernel(y_true.astype(jnp.int32), proxies)
